# baseline (speedup 1.0000x reference)
import functools

import jax
import jax.numpy as jnp
import numpy as np
from jax import lax
from jax.experimental import pallas as pl
from jax.experimental.pallas import tpu as pltpu
from jax.experimental.pallas import tpu_sc as plsc

BATCH = 1024
TOTAL_INPUT_BITS = 4096
NUM_NEURONS = 4096
N_BITS = 16
HASH_SIZE = 1024

N_EFF = 10
NUM_WORKERS = 32
NEURONS_PER_WORKER = NUM_NEURONS // NUM_WORKERS
GROUP = 8
NUM_GROUPS = NEURONS_PER_WORKER // GROUP
IDX_PER_GROUP = GROUP * N_EFF
LANES = 16
PACK = 4
WORDS = BATCH // PACK
NUM_CHUNKS = WORDS // LANES


def _sc_kernel_body(in_t, conn, mem, out, conn_v, idx_v, bits_v, mem_v,
                    out_v, sem_bits, sem_mem, sem_out):
    wid = lax.axis_index("s") * 2 + lax.axis_index("c")
    base_n = wid * NEURONS_PER_WORKER
    lane0 = jnp.zeros((LANES,), jnp.int32)

    pltpu.sync_copy(conn.at[pl.ds(base_n, NEURONS_PER_WORKER)], conn_v)
    iota = lax.iota(jnp.int32, LANES)

    def build_body(it, _):
        pos = it * LANES + iota
        j = lax.div(pos, N_EFF)
        k = pos - j * N_EFF
        vals = plsc.load_gather(conn_v, [j, k])
        idx_v[pl.ds(pl.multiple_of(it * LANES, LANES), LANES)] = vals
        return 0

    lax.fori_loop(0, NEURONS_PER_WORKER * N_EFF // LANES, build_body, 0)

    def start_group(g, parity):
        off = pl.multiple_of(g * IDX_PER_GROUP, 8)
        pltpu.async_copy(
            in_t.at[idx_v.at[pl.ds(off, IDX_PER_GROUP)]],
            bits_v.at[parity], sem_bits)
        pltpu.async_copy(
            mem.at[pl.ds(base_n + g * GROUP, GROUP)],
            mem_v.at[pl.ds(parity * GROUP, GROUP)], sem_mem)

    start_group(0, 0)

    def group_body(g, _):
        parity = lax.rem(g, 2)
        n0 = base_n + g * GROUP
        pltpu.make_async_copy(
            in_t.at[idx_v.at[pl.ds(0, IDX_PER_GROUP)]],
            bits_v.at[parity], sem_bits).wait()
        pltpu.make_async_copy(
            mem.at[pl.ds(0, GROUP)],
            mem_v.at[pl.ds(0, GROUP)], sem_mem).wait()

        @pl.when(g + 1 < NUM_GROUPS)
        def _():
            start_group(g + 1, 1 - parity)

        @pl.when(g >= 2)
        def _():
            pltpu.make_async_copy(
                out_v.at[parity], out.at[pl.ds(0, GROUP)], sem_out).wait()

        rows = [lane0 + (parity * GROUP + i) for i in range(GROUP)]

        def chunk_body(c, _):
            sl = pl.ds(pl.multiple_of(c * LANES, LANES), LANES)
            for i in range(GROUP):
                v = [
                    bits_v[parity, i * N_EFF + k, sl]
                    for k in range(N_EFF)
                ]
                t = [v[k] << k if k else v[0] for k in range(8)]
                while len(t) > 1:
                    t = [a + b for a, b in zip(t[::2], t[1::2])]
                s = t[0]
                packed = jnp.zeros((LANES,), jnp.int32)

                def hbit(vk, p, j):
                    d = p - 8 * j
                    x = (vk << d) if d >= 0 else (vk >> -d)
                    return x & (1 << p)

                for j in range(PACK):
                    lo = (s >> (8 * j)) & 0xFF
                    h8 = hbit(v[8], 8, j)
                    h9 = hbit(v[9], 9, j)
                    addr = lo | h8 | h9
                    val = plsc.load_gather(mem_v, [rows[i], addr]) & 1
                    packed = packed | (val << (8 * j))
                out_v[parity, i, sl] = packed
            return 0

        lax.fori_loop(0, NUM_CHUNKS, chunk_body, 0, unroll=8)
        pltpu.async_copy(out_v.at[parity], out.at[pl.ds(n0, GROUP)], sem_out)
        return 0

    lax.fori_loop(0, NUM_GROUPS, group_body, 0)
    pltpu.make_async_copy(
        out_v.at[0], out.at[pl.ds(0, GROUP)], sem_out).wait()
    pltpu.make_async_copy(
        out_v.at[1], out.at[pl.ds(0, GROUP)], sem_out).wait()


@jax.jit
def _run(in_t, connections, memory):
    mesh = plsc.VectorSubcoreMesh(core_axis_name="c", subcore_axis_name="s")
    f = pl.kernel(
        _sc_kernel_body,
        out_type=jax.ShapeDtypeStruct((NUM_NEURONS, WORDS), jnp.int32),
        mesh=mesh,
        compiler_params=pltpu.CompilerParams(needs_layout_passes=False),
        scratch_types=[
            pltpu.VMEM((NEURONS_PER_WORKER, N_BITS), jnp.int32),
            pltpu.VMEM((NEURONS_PER_WORKER * N_EFF,), jnp.int32),
            pltpu.VMEM((2, IDX_PER_GROUP, WORDS), jnp.int32),
            pltpu.VMEM((2 * GROUP, HASH_SIZE), jnp.int32),
            pltpu.VMEM((2, GROUP, WORDS), jnp.int32),
            pltpu.SemaphoreType.DMA,
            pltpu.SemaphoreType.DMA,
            pltpu.SemaphoreType.DMA,
        ],
    )
    return f(in_t, connections, memory)


import ml_dtypes

_SEL = np.zeros((2, BATCH, WORDS), np.float32)
for _b in range(BATCH):
    _SEL[(_b // WORDS) // 2, _b, _b % WORDS] = 256.0 ** ((_b // WORDS) % 2)
_S_LO = _SEL[0].astype(ml_dtypes.bfloat16)
_S_HI = _SEL[1].astype(ml_dtypes.bfloat16)

C_BLK = 4096
N_BLK = 4096
_DN = (((0,), (0,)), ((), ()))


def _pack_body(x_ref, slo_ref, shi_ref, o_ref):
    x = x_ref[...].astype(jnp.bfloat16)
    lo = lax.dot_general(x, slo_ref[...], _DN,
                         preferred_element_type=jnp.float32)
    hi = lax.dot_general(x, shi_ref[...], _DN,
                         preferred_element_type=jnp.float32)
    o_ref[...] = lo.astype(jnp.int32) + (hi.astype(jnp.int32) << 16)


def _unpack_body(x_ref, o_ref):
    xt = jnp.transpose(x_ref[...])
    planes = [((xt >> (8 * j)) & 1).astype(jnp.int8) for j in range(PACK)]
    o_ref[...] = jnp.concatenate(planes, axis=0)


@jax.jit
def _pack_tc(input_bits):
    return pl.pallas_call(
        _pack_body,
        grid=(TOTAL_INPUT_BITS // C_BLK,),
        in_specs=[
            pl.BlockSpec((BATCH, C_BLK), lambda i: (0, i)),
            pl.BlockSpec((BATCH, WORDS), lambda i: (0, 0)),
            pl.BlockSpec((BATCH, WORDS), lambda i: (0, 0)),
        ],
        out_specs=pl.BlockSpec((C_BLK, WORDS), lambda i: (i, 0)),
        out_shape=jax.ShapeDtypeStruct((TOTAL_INPUT_BITS, WORDS), jnp.int32),
    )(input_bits, _S_LO, _S_HI)


@jax.jit
def _unpack_tc(out_t):
    return pl.pallas_call(
        _unpack_body,
        grid=(NUM_NEURONS // N_BLK,),
        in_specs=[pl.BlockSpec((N_BLK, WORDS), lambda i: (i, 0))],
        out_specs=pl.BlockSpec((BATCH, N_BLK), lambda i: (0, i)),
        out_shape=jax.ShapeDtypeStruct((BATCH, NUM_NEURONS), jnp.int8),
    )(out_t)


def kernel(input_bits, connections, memory, bit_weights):
    del bit_weights
    in_t = _pack_tc(input_bits)
    out_t = _run(in_t, connections, memory)
    return _unpack_tc(out_t).astype(jnp.bool_)


# speedup vs baseline: 1.8859x; 1.8859x over previous
import functools

import jax
import jax.numpy as jnp
import numpy as np
from jax import lax
from jax.experimental import pallas as pl
from jax.experimental.pallas import tpu as pltpu
from jax.experimental.pallas import tpu_sc as plsc

BATCH = 1024
TOTAL_INPUT_BITS = 4096
NUM_NEURONS = 4096
N_BITS = 16
HASH_SIZE = 1024

N_EFF = 10
NUM_WORKERS = 32
NEURONS_PER_WORKER = NUM_NEURONS // NUM_WORKERS
GROUP = 8
NUM_GROUPS = NEURONS_PER_WORKER // GROUP
IDX_PER_GROUP = GROUP * N_EFF
LANES = 16
PACK = 4
WORDS = BATCH // PACK
NUM_CHUNKS = WORDS // LANES


def _sc_kernel_body(in_t, conn, mem, out, conn_v, idx_v, bits_v, mem_v,
                    out_v, sem_bits, sem_mem, sem_out):
    wid = lax.axis_index("s") * 2 + lax.axis_index("c")
    base_n = wid * NEURONS_PER_WORKER
    lane0 = jnp.zeros((LANES,), jnp.int32)

    pltpu.sync_copy(conn.at[pl.ds(base_n, NEURONS_PER_WORKER)], conn_v)
    iota = lax.iota(jnp.int32, LANES)

    def build_body(it, _):
        pos = it * LANES + iota
        j = lax.div(pos, N_EFF)
        k = pos - j * N_EFF
        vals = plsc.load_gather(conn_v, [j, k])
        idx_v[pl.ds(pl.multiple_of(it * LANES, LANES), LANES)] = vals
        return 0

    lax.fori_loop(0, NEURONS_PER_WORKER * N_EFF // LANES, build_body, 0)

    def start_group(g, parity):
        off = pl.multiple_of(g * IDX_PER_GROUP, 8)
        pltpu.async_copy(
            in_t.at[idx_v.at[pl.ds(off, IDX_PER_GROUP)]],
            bits_v.at[parity], sem_bits)
        pltpu.async_copy(
            mem.at[pl.ds(base_n + g * GROUP, GROUP)],
            mem_v.at[pl.ds(parity * GROUP, GROUP)], sem_mem)

    start_group(0, 0)

    def group_body(g, _):
        parity = lax.rem(g, 2)
        n0 = base_n + g * GROUP
        pltpu.make_async_copy(
            in_t.at[idx_v.at[pl.ds(0, IDX_PER_GROUP)]],
            bits_v.at[parity], sem_bits).wait()
        pltpu.make_async_copy(
            mem.at[pl.ds(0, GROUP)],
            mem_v.at[pl.ds(0, GROUP)], sem_mem).wait()

        @pl.when(g + 1 < NUM_GROUPS)
        def _():
            start_group(g + 1, 1 - parity)

        @pl.when(g >= 2)
        def _():
            pltpu.make_async_copy(
                out_v.at[parity], out.at[pl.ds(0, GROUP)], sem_out).wait()

        rows = [lane0 + (parity * GROUP + i) for i in range(GROUP)]

        def chunk_body(c, _):
            sl = pl.ds(pl.multiple_of(c * LANES, LANES), LANES)
            for i in range(GROUP):
                v = [
                    bits_v[parity, i * N_EFF + k, sl]
                    for k in range(N_EFF)
                ]
                t = [v[k] << k if k else v[0] for k in range(8)]
                while len(t) > 1:
                    t = [a + b for a, b in zip(t[::2], t[1::2])]
                s = t[0]
                packed = jnp.zeros((LANES,), jnp.int32)

                def hbit(vk, p, j):
                    d = p - 8 * j
                    x = (vk << d) if d >= 0 else (vk >> -d)
                    return x & (1 << p)

                for j in range(PACK):
                    lo = (s >> (8 * j)) & 0xFF
                    h8 = hbit(v[8], 8, j)
                    h9 = hbit(v[9], 9, j)
                    addr = lo | h8 | h9
                    val = plsc.load_gather(mem_v, [rows[i], addr]) & 1
                    packed = packed | (val << (8 * j))
                out_v[parity, i, sl] = packed
            return 0

        lax.fori_loop(0, NUM_CHUNKS, chunk_body, 0, unroll=4)
        pltpu.async_copy(out_v.at[parity], out.at[pl.ds(n0, GROUP)], sem_out)
        return 0

    lax.fori_loop(0, NUM_GROUPS, group_body, 0)
    pltpu.make_async_copy(
        out_v.at[0], out.at[pl.ds(0, GROUP)], sem_out).wait()
    pltpu.make_async_copy(
        out_v.at[1], out.at[pl.ds(0, GROUP)], sem_out).wait()


@jax.jit
def _run(in_t, connections, memory):
    mesh = plsc.VectorSubcoreMesh(core_axis_name="c", subcore_axis_name="s")
    f = pl.kernel(
        _sc_kernel_body,
        out_type=jax.ShapeDtypeStruct((NUM_NEURONS, WORDS), jnp.int32),
        mesh=mesh,
        compiler_params=pltpu.CompilerParams(needs_layout_passes=False),
        scratch_types=[
            pltpu.VMEM((NEURONS_PER_WORKER, N_BITS), jnp.int32),
            pltpu.VMEM((NEURONS_PER_WORKER * N_EFF,), jnp.int32),
            pltpu.VMEM((2, IDX_PER_GROUP, WORDS), jnp.int32),
            pltpu.VMEM((2 * GROUP, HASH_SIZE), jnp.int32),
            pltpu.VMEM((2, GROUP, WORDS), jnp.int32),
            pltpu.SemaphoreType.DMA,
            pltpu.SemaphoreType.DMA,
            pltpu.SemaphoreType.DMA,
        ],
    )
    return f(in_t, connections, memory)


import ml_dtypes

_SEL = np.zeros((2, BATCH, WORDS), np.float32)
for _b in range(BATCH):
    _SEL[(_b // WORDS) // 2, _b, _b % WORDS] = 256.0 ** ((_b // WORDS) % 2)
_S_LO = _SEL[0].astype(ml_dtypes.bfloat16)
_S_HI = _SEL[1].astype(ml_dtypes.bfloat16)

C_BLK = 2048
N_BLK = 2048
_DN = (((0,), (0,)), ((), ()))


def _pack_body(x_ref, slo_ref, shi_ref, o_ref):
    x = x_ref[...].astype(jnp.bfloat16)
    lo = lax.dot_general(x, slo_ref[...], _DN,
                         preferred_element_type=jnp.float32)
    hi = lax.dot_general(x, shi_ref[...], _DN,
                         preferred_element_type=jnp.float32)
    o_ref[...] = lo.astype(jnp.int32) + (hi.astype(jnp.int32) << 16)


def _unpack_body(x_ref, o_ref):
    xt = jnp.transpose(x_ref[...])
    planes = [((xt >> (8 * j)) & 1).astype(jnp.int8) for j in range(PACK)]
    o_ref[...] = jnp.concatenate(planes, axis=0)


@jax.jit
def _pack_tc(input_bits):
    return pl.pallas_call(
        _pack_body,
        grid=(TOTAL_INPUT_BITS // C_BLK,),
        in_specs=[
            pl.BlockSpec((BATCH, C_BLK), lambda i: (0, i)),
            pl.BlockSpec((BATCH, WORDS), lambda i: (0, 0)),
            pl.BlockSpec((BATCH, WORDS), lambda i: (0, 0)),
        ],
        out_specs=pl.BlockSpec((C_BLK, WORDS), lambda i: (i, 0)),
        out_shape=jax.ShapeDtypeStruct((TOTAL_INPUT_BITS, WORDS), jnp.int32),
    )(input_bits, _S_LO, _S_HI)


@jax.jit
def _unpack_tc(out_t):
    return pl.pallas_call(
        _unpack_body,
        grid=(NUM_NEURONS // N_BLK,),
        in_specs=[pl.BlockSpec((N_BLK, WORDS), lambda i: (i, 0))],
        out_specs=pl.BlockSpec((BATCH, N_BLK), lambda i: (0, i)),
        out_shape=jax.ShapeDtypeStruct((BATCH, NUM_NEURONS), jnp.int8),
    )(out_t)


def kernel(input_bits, connections, memory, bit_weights):
    del bit_weights
    in_t = _pack_tc(input_bits)
    out_t = _run(in_t, connections, memory)
    return _unpack_tc(out_t).astype(jnp.bool_)
